# Initial kernel scaffold; baseline (speedup 1.0000x reference)
import jax
import jax.numpy as jnp
from jax.experimental import pallas as pl


def kernel(x, emb0, emb1):
    raise NotImplementedError("write your pallas kernel here")


# speedup vs baseline: 1.3153x; 1.3153x over previous
import functools

import jax
import jax.numpy as jnp
from jax import lax
from jax.experimental import pallas as pl
from jax.experimental.pallas import tpu as pltpu
from jax.experimental.pallas import tpu_sc as plsc

_VOCAB = 100000
_DIM = 768
_SEQ = 8192

_NC = 2
_NS = 16
_NW = _NC * _NS
_ROWS_PER_W = _SEQ // _NW
_CHUNK = 64
_NCHUNKS = _ROWS_PER_W // _CHUNK
_LANES = 16
_VECS_PER_ROW = _DIM // _LANES


def _embed_body(x_hbm, emb0_hbm, emb1_hbm, out_hbm, idx_v, tok_v, pos_v, sem):
    wid = lax.axis_index("s") * _NC + lax.axis_index("c")
    base = wid * _ROWS_PER_W

    pltpu.sync_copy(x_hbm.at[wid], idx_v)

    for c in range(_NCHUNKS):
        row0 = base + c * _CHUNK
        gather = pltpu.async_copy(emb0_hbm.at[idx_v.at[c]], tok_v, sem)
        pltpu.sync_copy(emb1_hbm.at[pl.ds(row0, _CHUNK)], pos_v)
        gather.wait()

        def add_row(r, carry):
            for j in range(_VECS_PER_ROW):
                v = tok_v[r, pl.ds(j * _LANES, _LANES)]
                plsc.addupdate(pos_v.at[r, pl.ds(j * _LANES, _LANES)], v)
            return carry

        lax.fori_loop(0, _CHUNK, add_row, 0)

        pltpu.sync_copy(pos_v, out_hbm.at[pl.ds(row0, _CHUNK)])


@jax.jit
def _embed(x_grouped, emb0, emb1):
    mesh = plsc.VectorSubcoreMesh(core_axis_name="c", subcore_axis_name="s")
    run = functools.partial(
        pl.kernel,
        out_type=jax.ShapeDtypeStruct((_SEQ, _DIM), jnp.float32),
        mesh=mesh,
        scratch_types=[
            pltpu.VMEM((_NCHUNKS, _CHUNK), jnp.int32),
            pltpu.VMEM((_CHUNK, _DIM), jnp.float32),
            pltpu.VMEM((_CHUNK, _DIM), jnp.float32),
            pltpu.SemaphoreType.DMA,
        ],
    )(_embed_body)
    return run(x_grouped, emb0, emb1)


def kernel(x, emb0, emb1):
    x_grouped = x.reshape(_NW, _NCHUNKS, _CHUNK)
    return _embed(x_grouped, emb0, emb1)
